# Initial kernel scaffold; baseline (speedup 1.0000x reference)
import jax
import jax.numpy as jnp
from jax.experimental import pallas as pl


def kernel(cage, shape, shape_normals):
    raise NotImplementedError("write your pallas kernel here")


# speedup vs baseline: 1.1026x; 1.1026x over previous
import functools

import jax
import jax.numpy as jnp
from jax import lax
from jax.experimental import pallas as pl
from jax.experimental.pallas import tpu as pltpu
from jax.experimental.pallas import tpu_sc as plsc

EPS = 0.01
ITP = 10
PT = 256
NBLK = 2048

SC_NC = 2
SC_NS = 16
SC_LANES = 16
NW = SC_NC * SC_NS


def _knn_body(qb_ref, q2_ref, sT_ref, s2_ref, sx_ref, sy_ref, sz_ref,
              nx_ref, ny_ref, nz_ref, idx_ref, c_ref):
    b = pl.program_id(0)
    n = sx_ref.shape[-1]
    pt = q2_ref.shape[-1]

    sxr = sx_ref[0, 0, :]
    syr = sy_ref[0, 0, :]
    szr = sz_ref[0, 0, :]
    nxr = nx_ref[0, 0, :]
    nyr = ny_ref[0, 0, :]
    nzr = nz_ref[0, 0, :]
    c_ref[0, 0, :] = (sxr * nxr + syr * nyr + szr * nzr
                      + EPS * (nxr * nxr + nyr * nyr + nzr * nzr))

    qm = qb_ref[0]
    q2c = q2_ref[0, 0, :].reshape(pt, 1)

    run_min = jnp.full((pt,), jnp.inf, dtype=jnp.float32)
    run_idx = jnp.zeros((pt,), dtype=jnp.int32)
    for nb in range(n // NBLK):
        sl = pl.ds(nb * NBLK, NBLK)
        st = sT_ref[0, :, sl]
        s2b = s2_ref[0, 0, sl].reshape(1, NBLK)
        qs = jax.lax.dot_general(qm, st, (((1,), (0,)), ((), ())),
                                 preferred_element_type=jnp.float32)
        d2 = (q2c + s2b) - 2.0 * qs
        bmin = jnp.min(d2, axis=1)
        bidx = jnp.argmin(d2, axis=1).astype(jnp.int32)
        upd = bmin < run_min
        run_idx = jnp.where(upd, bidx + nb * NBLK, run_idx)
        run_min = jnp.where(upd, bmin, run_min)
    idx_ref[0, 0, :] = run_idx + b * n


def _sc_loss_body(qx_h, qy_h, qz_h, nx_h, ny_h, nz_h, c_h, idx_h, out_h,
                  idx_v, qx_v, qy_v, qz_v, nx_v, ny_v, nz_v, c_v, acc_v):
    qpw = idx_v.shape[0]
    wid = lax.axis_index("s") * SC_NC + lax.axis_index("c")
    base = wid * qpw
    pltpu.sync_copy(idx_h.at[pl.ds(base, qpw)], idx_v)
    pltpu.sync_copy(qx_h.at[pl.ds(base, qpw)], qx_v)
    pltpu.sync_copy(qy_h.at[pl.ds(base, qpw)], qy_v)
    pltpu.sync_copy(qz_h.at[pl.ds(base, qpw)], qz_v)
    pltpu.sync_copy(nx_h, nx_v)
    pltpu.sync_copy(ny_h, ny_v)
    pltpu.sync_copy(nz_h, nz_v)
    pltpu.sync_copy(c_h, c_v)

    acc = jnp.zeros((SC_LANES,), dtype=jnp.float32)
    for g in range(qpw // SC_LANES):
        sl = pl.ds(g * SC_LANES, SC_LANES)
        ii = idx_v[sl]
        gnx = plsc.load_gather(nx_v, [ii])
        gny = plsc.load_gather(ny_v, [ii])
        gnz = plsc.load_gather(nz_v, [ii])
        gc = plsc.load_gather(c_v, [ii])
        dot = qx_v[sl] * gnx + qy_v[sl] * gny + qz_v[sl] * gnz - gc
        acc = acc + jnp.maximum(-dot, 0.0)
    acc_v[...] = acc
    pltpu.sync_copy(acc_v, out_h.at[pl.ds(wid * SC_LANES, SC_LANES)])


def kernel(cage, shape, shape_normals):
    B, M, D = cage.shape
    N = shape.shape[1]
    P = M * ITP

    cage_p = jnp.roll(cage, shift=-1, axis=1)
    t = jnp.linspace(0.0, 1.0, ITP).reshape(1, 1, ITP, 1)
    q = (t * cage_p[:, :, None, :] + (1.0 - t) * cage[:, :, None, :])
    q = q.reshape(B, P, D)

    qb = q.astype(jnp.bfloat16)
    sT = shape.transpose(0, 2, 1).astype(jnp.bfloat16)
    q2 = jnp.sum(q * q, axis=-1).reshape(B, 1, P)
    s2 = jnp.sum(shape * shape, axis=-1).reshape(B, 1, N)

    qx = q[..., 0].reshape(B, 1, P)
    qy = q[..., 1].reshape(B, 1, P)
    qz = q[..., 2].reshape(B, 1, P)
    sx = shape[..., 0].reshape(B, 1, N)
    sy = shape[..., 1].reshape(B, 1, N)
    sz = shape[..., 2].reshape(B, 1, N)
    nx = shape_normals[..., 0].reshape(B, 1, N)
    ny = shape_normals[..., 1].reshape(B, 1, N)
    nz = shape_normals[..., 2].reshape(B, 1, N)

    s_spec = pl.BlockSpec((1, 1, N), lambda b, j: (b, 0, 0))
    nn_idx, cvals = pl.pallas_call(
        _knn_body,
        grid=(B, P // PT),
        in_specs=[pl.BlockSpec((1, PT, 3), lambda b, j: (b, j, 0)),
                  pl.BlockSpec((1, 1, PT), lambda b, j: (b, 0, j)),
                  pl.BlockSpec((1, 3, N), lambda b, j: (b, 0, 0)),
                  s_spec,
                  s_spec, s_spec, s_spec, s_spec, s_spec, s_spec],
        out_specs=[pl.BlockSpec((1, 1, PT), lambda b, j: (b, 0, j)),
                   pl.BlockSpec((1, 1, N), lambda b, j: (b, 0, 0))],
        out_shape=[jax.ShapeDtypeStruct((B, 1, P), jnp.int32),
                   jax.ShapeDtypeStruct((B, 1, N), jnp.float32)],
    )(qb, q2, sT, s2, sx, sy, sz, nx, ny, nz)

    qpw = (B * P) // NW
    sc_loss = functools.partial(
        pl.kernel,
        out_type=jax.ShapeDtypeStruct((NW * SC_LANES,), jnp.float32),
        mesh=plsc.VectorSubcoreMesh(core_axis_name="c", subcore_axis_name="s"),
        scratch_types=[
            pltpu.VMEM((qpw,), jnp.int32),
            pltpu.VMEM((qpw,), jnp.float32),
            pltpu.VMEM((qpw,), jnp.float32),
            pltpu.VMEM((qpw,), jnp.float32),
            pltpu.VMEM((B * N,), jnp.float32),
            pltpu.VMEM((B * N,), jnp.float32),
            pltpu.VMEM((B * N,), jnp.float32),
            pltpu.VMEM((B * N,), jnp.float32),
            pltpu.VMEM((SC_LANES,), jnp.float32),
        ],
        compiler_params=pltpu.CompilerParams(needs_layout_passes=False),
    )(_sc_loss_body)

    partials = sc_loss(
        qx.reshape(B * P), qy.reshape(B * P), qz.reshape(B * P),
        nx.reshape(B * N), ny.reshape(B * N), nz.reshape(B * N),
        cvals.reshape(B * N), nn_idx.reshape(B * P))
    return jnp.sum(partials) / (B * P)


# speedup vs baseline: 1.1820x; 1.0719x over previous
import functools

import jax
import jax.numpy as jnp
from jax import lax
from jax.experimental import pallas as pl
from jax.experimental.pallas import tpu as pltpu
from jax.experimental.pallas import tpu_sc as plsc

EPS = 0.01
ITP = 10
PT = 512
NBLK = 2048

SC_NC = 2
SC_NS = 16
SC_LANES = 16
NW = SC_NC * SC_NS


def _knn_body(qn2_ref, q2_ref, sT_ref, s2_ref, nT_ref, idx_ref, c_ref):
    b = pl.program_id(0)
    n = sT_ref.shape[-1]
    pt = q2_ref.shape[-1]

    sxr = sT_ref[0, 0, :]
    syr = sT_ref[0, 1, :]
    szr = sT_ref[0, 2, :]
    nxr = nT_ref[0, 0, :]
    nyr = nT_ref[0, 1, :]
    nzr = nT_ref[0, 2, :]
    c_ref[0, 0, :] = (sxr * nxr + syr * nyr + szr * nzr
                      + EPS * (nxr * nxr + nyr * nyr + nzr * nzr))

    qm = qn2_ref[0].astype(jnp.bfloat16)
    q2c = q2_ref[0, 0, :].reshape(pt, 1)

    run_min = jnp.full((pt,), jnp.inf, dtype=jnp.float32)
    run_idx = jnp.zeros((pt,), dtype=jnp.int32)
    for nb in range(n // NBLK):
        sl = pl.ds(nb * NBLK, NBLK)
        st = sT_ref[0, :, sl].astype(jnp.bfloat16)
        s2b = s2_ref[0, 0, sl].reshape(1, NBLK)
        qs2 = jax.lax.dot_general(qm, st, (((1,), (0,)), ((), ())),
                                  preferred_element_type=jnp.float32)
        d2 = (q2c + s2b) + qs2
        bmin = jnp.min(d2, axis=1)
        bidx = jnp.argmin(d2, axis=1).astype(jnp.int32)
        upd = bmin < run_min
        run_idx = jnp.where(upd, bidx + nb * NBLK, run_idx)
        run_min = jnp.where(upd, bmin, run_min)
    idx_ref[0, 0, :] = run_idx + b * n


def _sc_loss_body(q_h, n_h, c_h, idx_h, out_h, idx_v, q_v, n_v, c_v, acc_v):
    qpw = idx_v.shape[0]
    wid = lax.axis_index("s") * SC_NC + lax.axis_index("c")
    base = wid * qpw
    pltpu.sync_copy(idx_h.at[pl.ds(base, qpw)], idx_v)
    pltpu.sync_copy(q_h.at[pl.ds(base * 3, qpw * 3)], q_v)
    pltpu.sync_copy(n_h, n_v)
    pltpu.sync_copy(c_h, c_v)

    lane3 = jnp.arange(SC_LANES, dtype=jnp.int32) * 3
    acc = jnp.zeros((SC_LANES,), dtype=jnp.float32)
    for g in range(qpw // SC_LANES):
        ii = idx_v[pl.ds(g * SC_LANES, SC_LANES)]
        i3 = ii * 3
        gnx = plsc.load_gather(n_v, [i3])
        gny = plsc.load_gather(n_v, [i3 + 1])
        gnz = plsc.load_gather(n_v, [i3 + 2])
        gc = plsc.load_gather(c_v, [ii])
        q3 = lane3 + (g * SC_LANES * 3)
        qx = plsc.load_gather(q_v, [q3])
        qy = plsc.load_gather(q_v, [q3 + 1])
        qz = plsc.load_gather(q_v, [q3 + 2])
        dot = qx * gnx + qy * gny + qz * gnz - gc
        acc = acc + jnp.maximum(-dot, 0.0)
    acc_v[...] = acc
    pltpu.sync_copy(acc_v, out_h.at[pl.ds(wid * SC_LANES, SC_LANES)])


def kernel(cage, shape, shape_normals):
    B, M, D = cage.shape
    N = shape.shape[1]
    P = M * ITP

    cage_p = jnp.roll(cage, shift=-1, axis=1)
    t = jnp.linspace(0.0, 1.0, ITP).reshape(1, 1, ITP, 1)
    q = (t * cage_p[:, :, None, :] + (1.0 - t) * cage[:, :, None, :])
    q = q.reshape(B, P, D)

    qn2 = -2.0 * q
    q2 = jnp.sum(q * q, axis=-1).reshape(B, 1, P)
    sT = shape.transpose(0, 2, 1)
    s2 = jnp.sum(shape * shape, axis=-1).reshape(B, 1, N)
    nT = shape_normals.transpose(0, 2, 1)

    nn_idx, cvals = pl.pallas_call(
        _knn_body,
        grid=(B, P // PT),
        in_specs=[pl.BlockSpec((1, PT, 3), lambda b, j: (b, j, 0)),
                  pl.BlockSpec((1, 1, PT), lambda b, j: (b, 0, j)),
                  pl.BlockSpec((1, 3, N), lambda b, j: (b, 0, 0)),
                  pl.BlockSpec((1, 1, N), lambda b, j: (b, 0, 0)),
                  pl.BlockSpec((1, 3, N), lambda b, j: (b, 0, 0))],
        out_specs=[pl.BlockSpec((1, 1, PT), lambda b, j: (b, 0, j)),
                   pl.BlockSpec((1, 1, N), lambda b, j: (b, 0, 0))],
        out_shape=[jax.ShapeDtypeStruct((B, 1, P), jnp.int32),
                   jax.ShapeDtypeStruct((B, 1, N), jnp.float32)],
    )(qn2, q2, sT, s2, nT)

    qpw = (B * P) // NW
    sc_loss = functools.partial(
        pl.kernel,
        out_type=jax.ShapeDtypeStruct((NW * SC_LANES,), jnp.float32),
        mesh=plsc.VectorSubcoreMesh(core_axis_name="c", subcore_axis_name="s"),
        scratch_types=[
            pltpu.VMEM((qpw,), jnp.int32),
            pltpu.VMEM((qpw * 3,), jnp.float32),
            pltpu.VMEM((B * N * 3,), jnp.float32),
            pltpu.VMEM((B * N,), jnp.float32),
            pltpu.VMEM((SC_LANES,), jnp.float32),
        ],
        compiler_params=pltpu.CompilerParams(needs_layout_passes=False),
    )(_sc_loss_body)

    partials = sc_loss(
        q.reshape(B * P * D), shape_normals.reshape(B * N * D),
        cvals.reshape(B * N), nn_idx.reshape(B * P))
    return jnp.sum(partials) / (B * P)


# speedup vs baseline: 1.2343x; 1.0443x over previous
import functools

import jax
import jax.numpy as jnp
from jax import lax
from jax.experimental import pallas as pl
from jax.experimental.pallas import tpu as pltpu
from jax.experimental.pallas import tpu_sc as plsc

EPS = 0.01
ITP = 10
PT = 512
NBLK = 2048

SC_NC = 2
SC_NS = 16
SC_LANES = 16
NW = SC_NC * SC_NS


def _knn_body(qn2_ref, q2_ref, sT_ref, s2_ref, nT_ref, idx_ref, c_ref):
    b = pl.program_id(0)
    n = sT_ref.shape[-1]
    pt = q2_ref.shape[-1]

    sxr = sT_ref[0, 0, :]
    syr = sT_ref[0, 1, :]
    szr = sT_ref[0, 2, :]
    nxr = nT_ref[0, 0, :]
    nyr = nT_ref[0, 1, :]
    nzr = nT_ref[0, 2, :]
    c_ref[0, 0, :] = (sxr * nxr + syr * nyr + szr * nzr
                      + EPS * (nxr * nxr + nyr * nyr + nzr * nzr))

    qm = qn2_ref[0].astype(jnp.bfloat16)
    q2c = q2_ref[0, 0, :].reshape(pt, 1)

    run_min = jnp.full((pt,), jnp.inf, dtype=jnp.float32)
    run_idx = jnp.zeros((pt,), dtype=jnp.int32)
    for nb in range(n // NBLK):
        sl = pl.ds(nb * NBLK, NBLK)
        st = sT_ref[0, :, sl].astype(jnp.bfloat16)
        s2b = s2_ref[0, 0, sl].reshape(1, NBLK)
        qs2 = jax.lax.dot_general(qm, st, (((1,), (0,)), ((), ())),
                                  preferred_element_type=jnp.float32)
        d2 = (q2c + s2b) + qs2
        bmin = jnp.min(d2, axis=1)
        bidx = jnp.argmin(d2, axis=1).astype(jnp.int32)
        upd = bmin < run_min
        run_idx = jnp.where(upd, bidx + nb * NBLK, run_idx)
        run_min = jnp.where(upd, bmin, run_min)
    idx_ref[0, 0, :] = run_idx + b * n


def _sc_loss_body(q_h, n_h, c_h, idx_h, out_h, idx_v, q_v, n_v, c_v, acc_v):
    qpw = idx_v.shape[0]
    n1 = c_v.shape[0]
    wid = lax.axis_index("s") * SC_NC + lax.axis_index("c")
    base = wid * qpw
    batch = wid // SC_NS
    boff = batch * n1
    pltpu.sync_copy(idx_h.at[pl.ds(base, qpw)], idx_v)
    pltpu.sync_copy(q_h.at[pl.ds(base * 3, qpw * 3)], q_v)
    pltpu.sync_copy(n_h.at[pl.ds(boff * 3, n1 * 3)], n_v)
    pltpu.sync_copy(c_h.at[pl.ds(boff, n1)], c_v)

    lane3 = jnp.arange(SC_LANES, dtype=jnp.int32) * 3
    off3 = boff * 3
    acc = jnp.zeros((SC_LANES,), dtype=jnp.float32)
    for g in range(qpw // SC_LANES):
        ii = idx_v[pl.ds(g * SC_LANES, SC_LANES)]
        i3 = ii * 3 - off3
        gnx = plsc.load_gather(n_v, [i3])
        gny = plsc.load_gather(n_v, [i3 + 1])
        gnz = plsc.load_gather(n_v, [i3 + 2])
        gc = plsc.load_gather(c_v, [ii - boff])
        q3 = lane3 + (g * SC_LANES * 3)
        qx = plsc.load_gather(q_v, [q3])
        qy = plsc.load_gather(q_v, [q3 + 1])
        qz = plsc.load_gather(q_v, [q3 + 2])
        dot = qx * gnx + qy * gny + qz * gnz - gc
        acc = acc + jnp.maximum(-dot, 0.0)
    acc_v[...] = acc
    pltpu.sync_copy(acc_v, out_h.at[pl.ds(wid * SC_LANES, SC_LANES)])


def kernel(cage, shape, shape_normals):
    B, M, D = cage.shape
    N = shape.shape[1]
    P = M * ITP

    cage_p = jnp.roll(cage, shift=-1, axis=1)
    t = jnp.linspace(0.0, 1.0, ITP).reshape(1, 1, ITP, 1)
    q = (t * cage_p[:, :, None, :] + (1.0 - t) * cage[:, :, None, :])
    q = q.reshape(B, P, D)

    qn2 = -2.0 * q
    q2 = jnp.sum(q * q, axis=-1).reshape(B, 1, P)
    sT = shape.transpose(0, 2, 1)
    s2 = jnp.sum(shape * shape, axis=-1).reshape(B, 1, N)
    nT = shape_normals.transpose(0, 2, 1)

    nn_idx, cvals = pl.pallas_call(
        _knn_body,
        grid=(B, P // PT),
        in_specs=[pl.BlockSpec((1, PT, 3), lambda b, j: (b, j, 0)),
                  pl.BlockSpec((1, 1, PT), lambda b, j: (b, 0, j)),
                  pl.BlockSpec((1, 3, N), lambda b, j: (b, 0, 0)),
                  pl.BlockSpec((1, 1, N), lambda b, j: (b, 0, 0)),
                  pl.BlockSpec((1, 3, N), lambda b, j: (b, 0, 0))],
        out_specs=[pl.BlockSpec((1, 1, PT), lambda b, j: (b, 0, j)),
                   pl.BlockSpec((1, 1, N), lambda b, j: (b, 0, 0))],
        out_shape=[jax.ShapeDtypeStruct((B, 1, P), jnp.int32),
                   jax.ShapeDtypeStruct((B, 1, N), jnp.float32)],
    )(qn2, q2, sT, s2, nT)

    qpw = (B * P) // NW
    sc_loss = functools.partial(
        pl.kernel,
        out_type=jax.ShapeDtypeStruct((NW * SC_LANES,), jnp.float32),
        mesh=plsc.VectorSubcoreMesh(core_axis_name="c", subcore_axis_name="s"),
        scratch_types=[
            pltpu.VMEM((qpw,), jnp.int32),
            pltpu.VMEM((qpw * 3,), jnp.float32),
            pltpu.VMEM((N * 3,), jnp.float32),
            pltpu.VMEM((N,), jnp.float32),
            pltpu.VMEM((SC_LANES,), jnp.float32),
        ],
        compiler_params=pltpu.CompilerParams(needs_layout_passes=False),
    )(_sc_loss_body)

    partials = sc_loss(
        q.reshape(B * P * D), shape_normals.reshape(B * N * D),
        cvals.reshape(B * N), nn_idx.reshape(B * P))
    return jnp.sum(partials) / (B * P)
